# baseline (speedup 1.0000x reference)
import functools

import jax
import jax.numpy as jnp
from jax import lax
from jax.experimental import pallas as pl
from jax.experimental.pallas import tpu as pltpu
from jax.experimental.pallas import tpu_sc as plsc

_N = 10000
_D = 128
_NC = 2
_NS = 16
_NW = _NC * _NS
_CH = 128
_SUP = 1024
_S0 = 20
_S1 = 0
_NACC = 10112
_RPW = _NACC // _NS


@functools.lru_cache(maxsize=None)
def _seg_sum(k, e_pad):
    assert e_pad == _NS * (_S0 + _S1) * _SUP
    nch = _SUP // _CH
    mesh = plsc.VectorSubcoreMesh(core_axis_name="c", subcore_axis_name="s")

    def body(xt, packed, zeros, out, sup, srcv0, dstm0, srcv1, dstm1,
             rows0, rows1, acc, sg0, sg1, ss0, ss1):
        cid = lax.axis_index("c")
        sid = lax.axis_index("s")
        r0 = sid * _RPW
        trash = _N + sid

        def prep(base, srcv, dstm):
            for w in range(_CH // 16):
                sl = pl.ds(base + w * 16, 16)
                so = pl.ds(w * 16, 16)
                srcv[so] = sup[0, sl]
                dstm[so] = jnp.where(sup[2, sl] == k, sup[1, sl], trash)

        @pl.when(cid == 0)
        def _():
            pltpu.sync_copy(zeros, acc.at[pl.ds(r0, _RPW)])
            plsc.subcore_barrier()

            def pair(i, carry):
                base = i * 2 * _CH
                prep(base, srcv0, dstm0)
                g0 = pltpu.async_copy(xt.at[srcv0], rows0, sg0)
                prep(base + _CH, srcv1, dstm1)
                g1 = pltpu.async_copy(xt.at[srcv1], rows1, sg1)
                g0.wait()
                s0 = pltpu.async_copy(rows0, acc.at[dstm0], ss0, add=True)
                g1.wait()
                s1 = pltpu.async_copy(rows1, acc.at[dstm1], ss1, add=True)
                s0.wait()
                s1.wait()
                return carry

            def superchunk0(s, carry):
                pltpu.sync_copy(packed.at[sid * _S0 + s], sup)
                return lax.fori_loop(0, nch // 2, pair, carry)

            lax.fori_loop(0, _S0, superchunk0, 0)

            plsc.subcore_barrier()
            pltpu.sync_copy(acc.at[pl.ds(r0, _RPW)], out.at[pl.ds(r0, _RPW)])

    return pl.kernel(
        body,
        out_type=jax.ShapeDtypeStruct((_NACC, _D), jnp.float32),
        mesh=mesh,
        scratch_types=[
            pltpu.VMEM((3, _SUP), jnp.int32),
            pltpu.VMEM((_CH,), jnp.int32),
            pltpu.VMEM((_CH,), jnp.int32),
            pltpu.VMEM((_CH,), jnp.int32),
            pltpu.VMEM((_CH,), jnp.int32),
            pltpu.VMEM((_CH, _D), jnp.float32),
            pltpu.VMEM((_CH, _D), jnp.float32),
            pltpu.VMEM_SHARED((_NACC, _D), jnp.float32),
            pltpu.SemaphoreType.DMA,
            pltpu.SemaphoreType.DMA,
            pltpu.SemaphoreType.DMA,
            pltpu.SemaphoreType.DMA,
        ],
    )


@functools.lru_cache(maxsize=None)
def _tc_layer(nk, with_head, bn=1000):
    grid = (_N // bn,)
    dn = (((1,), (1,)), ((), ()))

    def body(*args):
        out_ref = args[-1]
        ps = args[:nk]
        xr = args[nk]
        w0 = nk + 1
        acc = jax.nn.relu(lax.dot_general(xr[...], args[w0 + 2 * nk][...], dn)
                          + args[w0 + 2 * nk + 1][...])
        for i in range(nk):
            agg = ps[i][...]
            acc = acc + jax.nn.relu(
                lax.dot_general(agg, args[w0 + 2 * i][...], dn)
                + args[w0 + 2 * i + 1][...])
        h = jax.nn.relu(acc)
        if with_head:
            out_ref[...] = (lax.dot_general(h, args[w0 + 2 * nk + 2][...], dn)
                            + args[w0 + 2 * nk + 3][...])
        else:
            out_ref[...] = h

    p_spec = pl.BlockSpec((bn, _D), lambda i: (i, 0))
    x_spec = pl.BlockSpec((bn, _D), lambda i: (i, 0))
    w_spec = pl.BlockSpec((_D, _D), lambda i: (0, 0))
    b_spec = pl.BlockSpec((1, _D), lambda i: (0, 0))
    n_wb = nk + 1 + (1 if with_head else 0)
    in_specs = ([p_spec] * nk + [x_spec] + [w_spec, b_spec] * n_wb)

    return pl.pallas_call(
        body,
        grid=grid,
        in_specs=in_specs,
        out_specs=pl.BlockSpec((bn, _D), lambda i: (i, 0)),
        out_shape=jax.ShapeDtypeStruct((_N, _D), jnp.float32),
    )


def kernel(x, edge_index, edge_attr, Ws_t0, bs_t0, Wk_t0_k1, bk_t0_k1,
           Ws_t1, bs_t1, Wk_t1_k1, bk_t1_k1, Wk_t1_k2, bk_t1_k2,
           Ws_t2, bs_t2, Wk_t2_k1, bk_t2_k1, Wk_t2_k2, bk_t2_k2,
           Wk_t2_k3, bk_t2_k3, Whead, bhead):
    e = edge_index.shape[1]
    quant = _NS * (_S0 + _S1) * _SUP
    e_pad = ((e + quant - 1) // quant) * quant
    pad = e_pad - e
    src = jnp.pad(edge_index[0], (0, pad))
    dst = jnp.pad(edge_index[1], (0, pad))
    attr = jnp.pad(edge_attr, (0, pad))
    packed = (jnp.stack([src, dst, attr])
              .reshape(3, e_pad // _SUP, _SUP).transpose(1, 0, 2))
    zeros = jnp.zeros((_RPW, _D), jnp.float32)

    def agg(xt, k):
        return _seg_sum(k, e_pad)(xt, packed, zeros)

    def rb(b):
        return b.reshape(1, _D)

    a01 = agg(x, 1)
    h1 = _tc_layer(1, False)(a01, x, Wk_t0_k1, rb(bk_t0_k1),
                             Ws_t0, rb(bs_t0))
    a11 = agg(h1, 1)
    a02 = agg(x, 2)
    h2 = _tc_layer(2, False)(a11, a02, h1, Wk_t1_k1, rb(bk_t1_k1),
                             Wk_t1_k2, rb(bk_t1_k2), Ws_t1, rb(bs_t1))
    a21 = agg(h2, 1)
    a12 = agg(h1, 2)
    a03 = agg(x, 3)
    return _tc_layer(3, True)(a21, a12, a03, h2,
                              Wk_t2_k1, rb(bk_t2_k1), Wk_t2_k2, rb(bk_t2_k2),
                              Wk_t2_k3, rb(bk_t2_k3), Ws_t2, rb(bs_t2),
                              Whead, rb(bhead))


# speedup vs baseline: 1.5998x; 1.5998x over previous
import functools

import jax
import jax.numpy as jnp
from jax import lax
from jax.experimental import pallas as pl
from jax.experimental.pallas import tpu as pltpu
from jax.experimental.pallas import tpu_sc as plsc

_N = 10000
_D = 128
_NC = 2
_NS = 16
_CH = 128
_SUP = 1024
_SPW = 20
_ZR = 64
_NACC = 10112
_RPW = _NACC // _NS


@functools.lru_cache(maxsize=None)
def _seg_pair(ka, kb, e_pad):
    assert e_pad == _NS * _SPW * _SUP
    nch = _SUP // _CH
    mesh = plsc.VectorSubcoreMesh(core_axis_name="c", subcore_axis_name="s")

    def body(xta, xtb, packed, zeros, out, sup, srcv0, dstm0, srcv1, dstm1,
             rows0, rows1, zbuf, acc, sg0, sg1, ss0, ss1):
        cid = lax.axis_index("c")
        sid = lax.axis_index("s")
        r0 = sid * _RPW
        trash = _N + sid

        def prep(base, kk, srcv, dstm):
            for w in range(_CH // 16):
                sl = pl.ds(base + w * 16, 16)
                so = pl.ds(w * 16, 16)
                srcv[so] = sup[0, sl]
                dstm[so] = jnp.where(sup[2, sl] == kk, sup[1, sl], trash)

        @pl.when(cid == 0)
        def _():
            pltpu.sync_copy(zeros, acc.at[pl.ds(r0, _RPW)])
            plsc.subcore_barrier()

            def pair(i, carry):
                base = i * 2 * _CH
                prep(base, ka, srcv0, dstm0)
                g0 = pltpu.async_copy(xta.at[srcv0], rows0, sg0)
                prep(base + _CH, ka, srcv1, dstm1)
                g1 = pltpu.async_copy(xta.at[srcv1], rows1, sg1)
                g0.wait()
                s0 = pltpu.async_copy(rows0, acc.at[dstm0], ss0, add=True)
                g1.wait()
                s1 = pltpu.async_copy(rows1, acc.at[dstm1], ss1, add=True)
                s0.wait()
                s1.wait()
                return carry

            def superchunk0(s, carry):
                pltpu.sync_copy(packed.at[sid * _SPW + s], sup)
                return lax.fori_loop(0, nch // 2, pair, carry)

            lax.fori_loop(0, _SPW, superchunk0, 0)

            plsc.subcore_barrier()
            pltpu.sync_copy(acc.at[pl.ds(r0, _RPW)],
                            out.at[0, pl.ds(r0, _RPW)])

        @pl.when(cid == 1)
        def _():
            def zfill(w, carry):
                slz = pl.ds(w * 16, 16)
                zv = jnp.zeros((16,), jnp.float32)
                for r in range(_ZR):
                    zbuf[r, slz] = zv
                return carry

            lax.fori_loop(0, _D // 16, zfill, 0)
            for j in range(_RPW // _ZR):
                pltpu.sync_copy(zbuf, acc.at[pl.ds(r0 + j * _ZR, _ZR)])
            rem = _RPW % _ZR
            if rem:
                pltpu.sync_copy(zbuf.at[pl.ds(0, rem)],
                                acc.at[pl.ds(r0 + _RPW - rem, rem)])
            plsc.subcore_barrier()

            def chunk(i, carry):
                base = i * _CH
                prep(base, kb, srcv0, dstm0)
                pltpu.async_copy(xtb.at[srcv0], rows0, sg0).wait()
                pltpu.sync_copy(rows0, acc.at[dstm0], add=True)
                return carry

            def superchunk1(s, carry):
                pltpu.sync_copy(packed.at[sid * _SPW + s], sup)
                return lax.fori_loop(0, nch, chunk, carry)

            lax.fori_loop(0, _SPW, superchunk1, 0)

            plsc.subcore_barrier()
            pltpu.sync_copy(acc.at[pl.ds(r0, _RPW)],
                            out.at[1, pl.ds(r0, _RPW)])

    return pl.kernel(
        body,
        out_type=jax.ShapeDtypeStruct((_NC, _NACC, _D), jnp.float32),
        mesh=mesh,
        scratch_types=[
            pltpu.VMEM((3, _SUP), jnp.int32),
            pltpu.VMEM((_CH,), jnp.int32),
            pltpu.VMEM((_CH,), jnp.int32),
            pltpu.VMEM((_CH,), jnp.int32),
            pltpu.VMEM((_CH,), jnp.int32),
            pltpu.VMEM((_CH, _D), jnp.float32),
            pltpu.VMEM((_CH, _D), jnp.float32),
            pltpu.VMEM((_ZR, _D), jnp.float32),
            pltpu.VMEM_SHARED((_NACC, _D), jnp.float32),
            pltpu.SemaphoreType.DMA,
            pltpu.SemaphoreType.DMA,
            pltpu.SemaphoreType.DMA,
            pltpu.SemaphoreType.DMA,
        ],
    )


@functools.lru_cache(maxsize=None)
def _tc_layer(nk, with_head, bn=1000):
    grid = (_N // bn,)
    dn = (((1,), (1,)), ((), ()))

    def body(*args):
        out_ref = args[-1]
        ps = args[:nk]
        xr = args[nk]
        w0 = nk + 1
        acc = jax.nn.relu(lax.dot_general(xr[...], args[w0 + 2 * nk][...], dn)
                          + args[w0 + 2 * nk + 1][...])
        for i in range(nk):
            agg = ps[i][0]
            acc = acc + jax.nn.relu(
                lax.dot_general(agg, args[w0 + 2 * i][...], dn)
                + args[w0 + 2 * i + 1][...])
        h = jax.nn.relu(acc)
        if with_head:
            out_ref[...] = (lax.dot_general(h, args[w0 + 2 * nk + 2][...], dn)
                            + args[w0 + 2 * nk + 3][...])
        else:
            out_ref[...] = h

    x_spec = pl.BlockSpec((bn, _D), lambda i: (i, 0))
    w_spec = pl.BlockSpec((_D, _D), lambda i: (0, 0))
    b_spec = pl.BlockSpec((1, _D), lambda i: (0, 0))
    n_wb = nk + 1 + (1 if with_head else 0)

    def build(slots):
        p_specs = [
            pl.BlockSpec((1, bn, _D), functools.partial(
                lambda s, i: (s, i, 0), slot))
            for slot in slots
        ]
        return pl.pallas_call(
            body,
            grid=grid,
            in_specs=(p_specs + [x_spec] + [w_spec, b_spec] * n_wb),
            out_specs=pl.BlockSpec((bn, _D), lambda i: (i, 0)),
            out_shape=jax.ShapeDtypeStruct((_N, _D), jnp.float32),
        )

    return build


def kernel(x, edge_index, edge_attr, Ws_t0, bs_t0, Wk_t0_k1, bk_t0_k1,
           Ws_t1, bs_t1, Wk_t1_k1, bk_t1_k1, Wk_t1_k2, bk_t1_k2,
           Ws_t2, bs_t2, Wk_t2_k1, bk_t2_k1, Wk_t2_k2, bk_t2_k2,
           Wk_t2_k3, bk_t2_k3, Whead, bhead):
    e = edge_index.shape[1]
    quant = _NS * _SPW * _SUP
    e_pad = ((e + quant - 1) // quant) * quant
    pad = e_pad - e
    src = jnp.pad(edge_index[0], (0, pad))
    dst = jnp.pad(edge_index[1], (0, pad))
    attr = jnp.pad(edge_attr, (0, pad))
    packed = (jnp.stack([src, dst, attr])
              .reshape(3, e_pad // _SUP, _SUP).transpose(1, 0, 2))
    zeros = jnp.zeros((_RPW, _D), jnp.float32)

    def aggpair(xta, ka, xtb, kb):
        return _seg_pair(ka, kb, e_pad)(xta, xtb, packed, zeros)

    def rb(b):
        return b.reshape(1, _D)

    c1 = aggpair(x, 1, x, 2)
    h1 = _tc_layer(1, False)([0])(c1, x, Wk_t0_k1, rb(bk_t0_k1),
                                  Ws_t0, rb(bs_t0))
    c2 = aggpair(h1, 1, x, 3)
    h2 = _tc_layer(2, False)([0, 1])(c2, c1, h1, Wk_t1_k1, rb(bk_t1_k1),
                                     Wk_t1_k2, rb(bk_t1_k2),
                                     Ws_t1, rb(bs_t1))
    c3 = aggpair(h2, 1, h1, 2)
    return _tc_layer(3, True)([0, 1, 1])(c3, c3, c2, h2,
                                         Wk_t2_k1, rb(bk_t2_k1),
                                         Wk_t2_k2, rb(bk_t2_k2),
                                         Wk_t2_k3, rb(bk_t2_k3),
                                         Ws_t2, rb(bs_t2),
                                         Whead, rb(bhead))


# speedup vs baseline: 1.6829x; 1.0519x over previous
import functools

import jax
import jax.numpy as jnp
from jax import lax
from jax.experimental import pallas as pl
from jax.experimental.pallas import tpu as pltpu
from jax.experimental.pallas import tpu_sc as plsc

_N = 10000
_D = 128
_NC = 2
_NS = 16
_CH = 128
_SUP = 1024
_SPW = 20
_ZR = 64
_NACC = 10112
_RPW = _NACC // _NS


@functools.lru_cache(maxsize=None)
def _seg_pair(ka, kb, e_pad):
    assert e_pad == _NS * _SPW * _SUP
    nch = _SUP // _CH
    mesh = plsc.VectorSubcoreMesh(core_axis_name="c", subcore_axis_name="s")

    def body(xta, xtb, packed, zeros, out, sup, srcv0, dstm0, srcv1, dstm1,
             rows0, rows1, zbuf, acc, sg0, sg1, ss0, ss1):
        cid = lax.axis_index("c")
        sid = lax.axis_index("s")
        r0 = sid * _RPW
        trash = _N + sid

        def prep(base, kk, srcv, dstm):
            for w in range(_CH // 16):
                sl = pl.ds(base + w * 16, 16)
                so = pl.ds(w * 16, 16)
                srcv[so] = sup[0, sl]
                dstm[so] = jnp.where(sup[2, sl] == kk, sup[1, sl], trash)

        @pl.when(cid == 0)
        def _():
            pltpu.sync_copy(zeros, acc.at[pl.ds(r0, _RPW)])
            plsc.subcore_barrier()

            def pair(i, carry):
                base = i * 2 * _CH
                prep(base, ka, srcv0, dstm0)
                g0 = pltpu.async_copy(xta.at[srcv0], rows0, sg0)
                prep(base + _CH, ka, srcv1, dstm1)
                g1 = pltpu.async_copy(xta.at[srcv1], rows1, sg1)
                g0.wait()
                s0 = pltpu.async_copy(rows0, acc.at[dstm0], ss0, add=True)
                g1.wait()
                s1 = pltpu.async_copy(rows1, acc.at[dstm1], ss1, add=True)
                s0.wait()
                s1.wait()
                return carry

            def superchunk0(s, carry):
                pltpu.sync_copy(packed.at[sid * _SPW + s], sup)
                return lax.fori_loop(0, nch // 2, pair, carry)

            lax.fori_loop(0, _SPW, superchunk0, 0)

            plsc.subcore_barrier()
            pltpu.sync_copy(acc.at[pl.ds(r0, _RPW)],
                            out.at[0, pl.ds(r0, _RPW)])

        @pl.when(cid == 1)
        def _():
            def zfill(w, carry):
                slz = pl.ds(w * 16, 16)
                zv = jnp.zeros((16,), jnp.float32)
                for r in range(_ZR):
                    zbuf[r, slz] = zv
                return carry

            lax.fori_loop(0, _D // 16, zfill, 0)
            for j in range(_RPW // _ZR):
                pltpu.sync_copy(zbuf, acc.at[pl.ds(r0 + j * _ZR, _ZR)])
            rem = _RPW % _ZR
            if rem:
                pltpu.sync_copy(zbuf.at[pl.ds(0, rem)],
                                acc.at[pl.ds(r0 + _RPW - rem, rem)])
            plsc.subcore_barrier()

            def pairb(i, carry):
                base = i * 2 * _CH
                prep(base, kb, srcv0, dstm0)
                g0 = pltpu.async_copy(xtb.at[srcv0], rows0, sg0)
                prep(base + _CH, kb, srcv1, dstm1)
                g1 = pltpu.async_copy(xtb.at[srcv1], rows1, sg1)
                g0.wait()
                s0 = pltpu.async_copy(rows0, acc.at[dstm0], ss0, add=True)
                g1.wait()
                s1 = pltpu.async_copy(rows1, acc.at[dstm1], ss1, add=True)
                s0.wait()
                s1.wait()
                return carry

            def superchunk1(s, carry):
                pltpu.sync_copy(packed.at[sid * _SPW + s], sup)
                return lax.fori_loop(0, nch // 2, pairb, carry)

            lax.fori_loop(0, _SPW, superchunk1, 0)

            plsc.subcore_barrier()
            pltpu.sync_copy(acc.at[pl.ds(r0, _RPW)],
                            out.at[1, pl.ds(r0, _RPW)])

    return pl.kernel(
        body,
        out_type=jax.ShapeDtypeStruct((_NC, _NACC, _D), jnp.float32),
        mesh=mesh,
        scratch_types=[
            pltpu.VMEM((3, _SUP), jnp.int32),
            pltpu.VMEM((_CH,), jnp.int32),
            pltpu.VMEM((_CH,), jnp.int32),
            pltpu.VMEM((_CH,), jnp.int32),
            pltpu.VMEM((_CH,), jnp.int32),
            pltpu.VMEM((_CH, _D), jnp.float32),
            pltpu.VMEM((_CH, _D), jnp.float32),
            pltpu.VMEM((_ZR, _D), jnp.float32),
            pltpu.VMEM_SHARED((_NACC, _D), jnp.float32),
            pltpu.SemaphoreType.DMA,
            pltpu.SemaphoreType.DMA,
            pltpu.SemaphoreType.DMA,
            pltpu.SemaphoreType.DMA,
        ],
    )


@functools.lru_cache(maxsize=None)
def _tc_layer(nk, with_head, bn=1000):
    grid = (_N // bn,)
    dn = (((1,), (1,)), ((), ()))

    def body(*args):
        out_ref = args[-1]
        ps = args[:nk]
        xr = args[nk]
        w0 = nk + 1
        acc = jax.nn.relu(lax.dot_general(xr[...], args[w0 + 2 * nk][...], dn)
                          + args[w0 + 2 * nk + 1][...])
        for i in range(nk):
            agg = ps[i][0]
            acc = acc + jax.nn.relu(
                lax.dot_general(agg, args[w0 + 2 * i][...], dn)
                + args[w0 + 2 * i + 1][...])
        h = jax.nn.relu(acc)
        if with_head:
            out_ref[...] = (lax.dot_general(h, args[w0 + 2 * nk + 2][...], dn)
                            + args[w0 + 2 * nk + 3][...])
        else:
            out_ref[...] = h

    x_spec = pl.BlockSpec((bn, _D), lambda i: (i, 0))
    w_spec = pl.BlockSpec((_D, _D), lambda i: (0, 0))
    b_spec = pl.BlockSpec((1, _D), lambda i: (0, 0))
    n_wb = nk + 1 + (1 if with_head else 0)

    def build(slots):
        p_specs = [
            pl.BlockSpec((1, bn, _D), functools.partial(
                lambda s, i: (s, i, 0), slot))
            for slot in slots
        ]
        return pl.pallas_call(
            body,
            grid=grid,
            in_specs=(p_specs + [x_spec] + [w_spec, b_spec] * n_wb),
            out_specs=pl.BlockSpec((bn, _D), lambda i: (i, 0)),
            out_shape=jax.ShapeDtypeStruct((_N, _D), jnp.float32),
        )

    return build


def kernel(x, edge_index, edge_attr, Ws_t0, bs_t0, Wk_t0_k1, bk_t0_k1,
           Ws_t1, bs_t1, Wk_t1_k1, bk_t1_k1, Wk_t1_k2, bk_t1_k2,
           Ws_t2, bs_t2, Wk_t2_k1, bk_t2_k1, Wk_t2_k2, bk_t2_k2,
           Wk_t2_k3, bk_t2_k3, Whead, bhead):
    e = edge_index.shape[1]
    quant = _NS * _SPW * _SUP
    e_pad = ((e + quant - 1) // quant) * quant
    pad = e_pad - e
    src = jnp.pad(edge_index[0], (0, pad))
    dst = jnp.pad(edge_index[1], (0, pad))
    attr = jnp.pad(edge_attr, (0, pad))
    packed = (jnp.stack([src, dst, attr])
              .reshape(3, e_pad // _SUP, _SUP).transpose(1, 0, 2))
    zeros = jnp.zeros((_RPW, _D), jnp.float32)

    def aggpair(xta, ka, xtb, kb):
        return _seg_pair(ka, kb, e_pad)(xta, xtb, packed, zeros)

    def rb(b):
        return b.reshape(1, _D)

    c1 = aggpair(x, 1, x, 2)
    h1 = _tc_layer(1, False)([0])(c1, x, Wk_t0_k1, rb(bk_t0_k1),
                                  Ws_t0, rb(bs_t0))
    c2 = aggpair(h1, 1, x, 3)
    h2 = _tc_layer(2, False)([0, 1])(c2, c1, h1, Wk_t1_k1, rb(bk_t1_k1),
                                     Wk_t1_k2, rb(bk_t1_k2),
                                     Ws_t1, rb(bs_t1))
    c3 = aggpair(h2, 1, h1, 2)
    return _tc_layer(3, True)([0, 1, 1])(c3, c3, c2, h2,
                                         Wk_t2_k1, rb(bk_t2_k1),
                                         Wk_t2_k2, rb(bk_t2_k2),
                                         Wk_t2_k3, rb(bk_t2_k3),
                                         Ws_t2, rb(bs_t2),
                                         Whead, rb(bhead))


# speedup vs baseline: 1.7035x; 1.0123x over previous
import functools

import jax
import jax.numpy as jnp
from jax import lax
from jax.experimental import pallas as pl
from jax.experimental.pallas import tpu as pltpu
from jax.experimental.pallas import tpu_sc as plsc

_N = 10000
_D = 128
_NC = 2
_NS = 16
_CH = 128
_SUP = 2048
_SPW = 10
_ZR = 64
_NACC = 10112
_RPW = _NACC // _NS


@functools.lru_cache(maxsize=None)
def _seg_pair(ka, kb, e_pad):
    assert e_pad == _NS * _SPW * _SUP
    nch = _SUP // _CH
    mesh = plsc.VectorSubcoreMesh(core_axis_name="c", subcore_axis_name="s")

    def body(xta, xtb, packed, zeros, out, sup, srcv0, dstm0, srcv1, dstm1,
             rows0, rows1, zbuf, acc, sg0, sg1, ss0, ss1):
        cid = lax.axis_index("c")
        sid = lax.axis_index("s")
        r0 = sid * _RPW
        trash = _N + sid

        def prep(base, kk, srcv, dstm):
            for w in range(_CH // 16):
                sl = pl.ds(base + w * 16, 16)
                so = pl.ds(w * 16, 16)
                srcv[so] = sup[0, sl]
                dstm[so] = jnp.where(sup[2, sl] == kk, sup[1, sl], trash)

        @pl.when(cid == 0)
        def _():
            pltpu.sync_copy(zeros, acc.at[pl.ds(r0, _RPW)])
            plsc.subcore_barrier()

            def pair(i, carry):
                base = i * 2 * _CH
                prep(base, ka, srcv0, dstm0)
                g0 = pltpu.async_copy(xta.at[srcv0], rows0, sg0)
                prep(base + _CH, ka, srcv1, dstm1)
                g1 = pltpu.async_copy(xta.at[srcv1], rows1, sg1)
                g0.wait()
                s0 = pltpu.async_copy(rows0, acc.at[dstm0], ss0, add=True)
                g1.wait()
                s1 = pltpu.async_copy(rows1, acc.at[dstm1], ss1, add=True)
                s0.wait()
                s1.wait()
                return carry

            def superchunk0(s, carry):
                pltpu.sync_copy(packed.at[sid * _SPW + s], sup)
                return lax.fori_loop(0, nch // 2, pair, carry)

            lax.fori_loop(0, _SPW, superchunk0, 0)

            plsc.subcore_barrier()
            pltpu.sync_copy(acc.at[pl.ds(r0, _RPW)],
                            out.at[0, pl.ds(r0, _RPW)])

        @pl.when(cid == 1)
        def _():
            def zfill(w, carry):
                slz = pl.ds(w * 16, 16)
                zv = jnp.zeros((16,), jnp.float32)
                for r in range(_ZR):
                    zbuf[r, slz] = zv
                return carry

            lax.fori_loop(0, _D // 16, zfill, 0)
            for j in range(_RPW // _ZR):
                pltpu.sync_copy(zbuf, acc.at[pl.ds(r0 + j * _ZR, _ZR)])
            rem = _RPW % _ZR
            if rem:
                pltpu.sync_copy(zbuf.at[pl.ds(0, rem)],
                                acc.at[pl.ds(r0 + _RPW - rem, rem)])
            plsc.subcore_barrier()

            def pairb(i, carry):
                base = i * 2 * _CH
                prep(base, kb, srcv0, dstm0)
                g0 = pltpu.async_copy(xtb.at[srcv0], rows0, sg0)
                prep(base + _CH, kb, srcv1, dstm1)
                g1 = pltpu.async_copy(xtb.at[srcv1], rows1, sg1)
                g0.wait()
                s0 = pltpu.async_copy(rows0, acc.at[dstm0], ss0, add=True)
                g1.wait()
                s1 = pltpu.async_copy(rows1, acc.at[dstm1], ss1, add=True)
                s0.wait()
                s1.wait()
                return carry

            def superchunk1(s, carry):
                pltpu.sync_copy(packed.at[sid * _SPW + s], sup)
                return lax.fori_loop(0, nch // 2, pairb, carry)

            lax.fori_loop(0, _SPW, superchunk1, 0)

            plsc.subcore_barrier()
            pltpu.sync_copy(acc.at[pl.ds(r0, _RPW)],
                            out.at[1, pl.ds(r0, _RPW)])

    return pl.kernel(
        body,
        out_type=jax.ShapeDtypeStruct((_NC, _NACC, _D), jnp.float32),
        mesh=mesh,
        scratch_types=[
            pltpu.VMEM((3, _SUP), jnp.int32),
            pltpu.VMEM((_CH,), jnp.int32),
            pltpu.VMEM((_CH,), jnp.int32),
            pltpu.VMEM((_CH,), jnp.int32),
            pltpu.VMEM((_CH,), jnp.int32),
            pltpu.VMEM((_CH, _D), jnp.float32),
            pltpu.VMEM((_CH, _D), jnp.float32),
            pltpu.VMEM((_ZR, _D), jnp.float32),
            pltpu.VMEM_SHARED((_NACC, _D), jnp.float32),
            pltpu.SemaphoreType.DMA,
            pltpu.SemaphoreType.DMA,
            pltpu.SemaphoreType.DMA,
            pltpu.SemaphoreType.DMA,
        ],
    )


@functools.lru_cache(maxsize=None)
def _tc_layer(nk, with_head, bn=1000):
    grid = (_N // bn,)
    dn = (((1,), (1,)), ((), ()))

    def body(*args):
        out_ref = args[-1]
        ps = args[:nk]
        xr = args[nk]
        w0 = nk + 1
        acc = jax.nn.relu(lax.dot_general(xr[...], args[w0 + 2 * nk][...], dn)
                          + args[w0 + 2 * nk + 1][...])
        for i in range(nk):
            agg = ps[i][0]
            acc = acc + jax.nn.relu(
                lax.dot_general(agg, args[w0 + 2 * i][...], dn)
                + args[w0 + 2 * i + 1][...])
        h = jax.nn.relu(acc)
        if with_head:
            out_ref[...] = (lax.dot_general(h, args[w0 + 2 * nk + 2][...], dn)
                            + args[w0 + 2 * nk + 3][...])
        else:
            out_ref[...] = h

    x_spec = pl.BlockSpec((bn, _D), lambda i: (i, 0))
    w_spec = pl.BlockSpec((_D, _D), lambda i: (0, 0))
    b_spec = pl.BlockSpec((1, _D), lambda i: (0, 0))
    n_wb = nk + 1 + (1 if with_head else 0)

    def build(slots):
        p_specs = [
            pl.BlockSpec((1, bn, _D), functools.partial(
                lambda s, i: (s, i, 0), slot))
            for slot in slots
        ]
        return pl.pallas_call(
            body,
            grid=grid,
            in_specs=(p_specs + [x_spec] + [w_spec, b_spec] * n_wb),
            out_specs=pl.BlockSpec((bn, _D), lambda i: (i, 0)),
            out_shape=jax.ShapeDtypeStruct((_N, _D), jnp.float32),
        )

    return build


def kernel(x, edge_index, edge_attr, Ws_t0, bs_t0, Wk_t0_k1, bk_t0_k1,
           Ws_t1, bs_t1, Wk_t1_k1, bk_t1_k1, Wk_t1_k2, bk_t1_k2,
           Ws_t2, bs_t2, Wk_t2_k1, bk_t2_k1, Wk_t2_k2, bk_t2_k2,
           Wk_t2_k3, bk_t2_k3, Whead, bhead):
    e = edge_index.shape[1]
    quant = _NS * _SPW * _SUP
    e_pad = ((e + quant - 1) // quant) * quant
    pad = e_pad - e
    src = jnp.pad(edge_index[0], (0, pad))
    dst = jnp.pad(edge_index[1], (0, pad))
    attr = jnp.pad(edge_attr, (0, pad))
    packed = (jnp.stack([src, dst, attr])
              .reshape(3, e_pad // _SUP, _SUP).transpose(1, 0, 2))
    zeros = jnp.zeros((_RPW, _D), jnp.float32)

    def aggpair(xta, ka, xtb, kb):
        return _seg_pair(ka, kb, e_pad)(xta, xtb, packed, zeros)

    def rb(b):
        return b.reshape(1, _D)

    c1 = aggpair(x, 1, x, 2)
    h1 = _tc_layer(1, False)([0])(c1, x, Wk_t0_k1, rb(bk_t0_k1),
                                  Ws_t0, rb(bs_t0))
    c2 = aggpair(h1, 1, x, 3)
    h2 = _tc_layer(2, False)([0, 1])(c2, c1, h1, Wk_t1_k1, rb(bk_t1_k1),
                                     Wk_t1_k2, rb(bk_t1_k2),
                                     Ws_t1, rb(bs_t1))
    c3 = aggpair(h2, 1, h1, 2)
    return _tc_layer(3, True)([0, 1, 1])(c3, c3, c2, h2,
                                         Wk_t2_k1, rb(bk_t2_k1),
                                         Wk_t2_k2, rb(bk_t2_k2),
                                         Wk_t2_k3, rb(bk_t2_k3),
                                         Ws_t2, rb(bs_t2),
                                         Whead, rb(bhead))
